# Initial kernel scaffold; baseline (speedup 1.0000x reference)
import jax
import jax.numpy as jnp
from jax.experimental import pallas as pl


def kernel(x, edge_index, edge_weight, W1, b1, W2, b2, temp):
    raise NotImplementedError("write your pallas kernel here")


# speedup vs baseline: 5.2988x; 5.2988x over previous
import functools
import jax
import jax.numpy as jnp
from jax import lax
from jax.experimental import pallas as pl
from jax.experimental.pallas import tpu as pltpu
from jax.experimental.pallas import tpu_sc as plsc

NC = 2
NS = 16
NW = NC * NS
EDGE_B = 128
N_PAD = 10240
D = 128
ROWS_PER_TILE = N_PAD // NS

_mesh = functools.partial(
    plsc.VectorSubcoreMesh, core_axis_name="c", subcore_axis_name="s",
    num_cores=NC, num_subcores=NS)


def _wid():
  return lax.axis_index("s") * NC + lax.axis_index("c")


def _make_deg(e_per_w):
  batches = e_per_w // EDGE_B

  DW = 16

  @functools.partial(
      pl.kernel,
      out_type=jax.ShapeDtypeStruct((NC, N_PAD, DW), jnp.float32),
      mesh=_mesh(),
      scratch_types=[
          pltpu.VMEM((EDGE_B,), jnp.int32),
          pltpu.VMEM((EDGE_B,), jnp.float32),
          pltpu.VMEM((EDGE_B, DW), jnp.float32),
          pltpu.VMEM_SHARED((N_PAD, DW), jnp.float32),
      ],
  )
  def k(dst_hbm, ew_hbm, out_hbm, idx_v, ew_v, ew_col, deg_sh):
    cid = lax.axis_index("c")
    sid = lax.axis_index("s")
    wid = _wid()
    row0 = sid * ROWS_PER_TILE

    def zb(i, _):
      ew_col[i, pl.ds(0, 16)] = jnp.zeros((16,), jnp.float32)
      return 0
    lax.fori_loop(0, EDGE_B, zb, 0)
    for t in range(ROWS_PER_TILE // EDGE_B):
      pltpu.sync_copy(ew_col, deg_sh.at[pl.ds(row0 + t * EDGE_B, EDGE_B)])
    plsc.subcore_barrier()

    def ebody(b, _):
      ebase = wid * e_per_w + b * EDGE_B
      pltpu.sync_copy(dst_hbm.at[pl.ds(ebase, EDGE_B)], idx_v)
      pltpu.sync_copy(ew_hbm.at[pl.ds(ebase, EDGE_B)], ew_v)

      def sbody(g, _):
        w16 = ew_v[pl.ds(g * 16, 16)]
        base = g * 16
        for l in range(16):
          ew_col[base + l, pl.ds(0, 16)] = jnp.full((16,), w16[l],
                                                    jnp.float32)
        return 0
      lax.fori_loop(0, EDGE_B // 16, sbody, 0)

      pltpu.sync_copy(ew_col, deg_sh.at[idx_v], add=True)
      return 0
    lax.fori_loop(0, batches, ebody, 0)

    plsc.subcore_barrier()
    pltpu.sync_copy(deg_sh.at[pl.ds(row0, ROWS_PER_TILE)],
                    out_hbm.at[cid, pl.ds(row0, ROWS_PER_TILE)])

  return k


def _make_prop(e_per_w):
  batches = e_per_w // EDGE_B

  @functools.partial(
      pl.kernel,
      out_type=jax.ShapeDtypeStruct((NC, N_PAD, D), jnp.float32),
      mesh=_mesh(),
      scratch_types=[
          pltpu.VMEM((EDGE_B,), jnp.int32),
          pltpu.VMEM((EDGE_B,), jnp.int32),
          pltpu.VMEM((EDGE_B,), jnp.float32),
          pltpu.VMEM((EDGE_B, D), jnp.float32),
          pltpu.VMEM_SHARED((N_PAD, D), jnp.float32),
          pltpu.SemaphoreType.DMA,
      ],
  )
  def k(hp_hbm, src_hbm, dst_hbm, ew_hbm, out_hbm,
        src_v, dst_v, ew_v, rows_v, acc_sh, sem):
    cid = lax.axis_index("c")
    sid = lax.axis_index("s")
    wid = _wid()
    row0 = sid * ROWS_PER_TILE

    def zb(i, _):
      for j in range(D // 16):
        rows_v[i, pl.ds(j * 16, 16)] = jnp.zeros((16,), jnp.float32)
      return 0
    lax.fori_loop(0, EDGE_B, zb, 0)
    for t in range(ROWS_PER_TILE // EDGE_B):
      pltpu.sync_copy(rows_v, acc_sh.at[pl.ds(row0 + t * EDGE_B, EDGE_B)])
    plsc.subcore_barrier()

    def ebody(b, _):
      ebase = wid * e_per_w + b * EDGE_B
      pltpu.sync_copy(src_hbm.at[pl.ds(ebase, EDGE_B)], src_v)
      pltpu.sync_copy(dst_hbm.at[pl.ds(ebase, EDGE_B)], dst_v)
      pltpu.sync_copy(ew_hbm.at[pl.ds(ebase, EDGE_B)], ew_v)
      pltpu.async_copy(hp_hbm.at[src_v], rows_v, sem).wait()

      def sbody(g, _):
        w16 = ew_v[pl.ds(g * 16, 16)]
        base = g * 16
        for l in range(16):
          w = w16[l]
          for j in range(D // 16):
            rows_v[base + l, pl.ds(j * 16, 16)] = (
                rows_v[base + l, pl.ds(j * 16, 16)] * w)
        return 0
      lax.fori_loop(0, EDGE_B // 16, sbody, 0)

      pltpu.sync_copy(rows_v, acc_sh.at[dst_v], add=True)
      return 0
    lax.fori_loop(0, batches, ebody, 0)

    plsc.subcore_barrier()
    pltpu.sync_copy(acc_sh.at[pl.ds(row0, ROWS_PER_TILE)],
                    out_hbm.at[cid, pl.ds(row0, ROWS_PER_TILE)])

  return k


_RB = 1024
_GRID = N_PAD // _RB


def _mlp_body(x_ref, w1_ref, b1_ref, w2_ref, b2_ref, degp_ref, t0_ref,
              hp_ref, hid_ref, dis_ref):
  xb = x_ref[...]
  h = jnp.maximum(jnp.dot(xb, w1_ref[...],
                          preferred_element_type=jnp.float32) + b1_ref[...], 0.0)
  h = jnp.dot(h, w2_ref[...], preferred_element_type=jnp.float32) + b2_ref[...]
  deg = jnp.sum(degp_ref[...], axis=0) + 1.0
  dis = lax.rsqrt(deg)
  hid_ref[...] = t0_ref[0, 0] * h
  hp_ref[...] = dis * h
  dis_ref[...] = dis


def _mlp_call(x_pad, W1, b1, W2, b2, degp3, t0):
  return pl.pallas_call(
      _mlp_body,
      grid=(_GRID,),
      in_specs=[
          pl.BlockSpec((_RB, D), lambda i: (i, 0)),
          pl.BlockSpec((D, D), lambda i: (0, 0)),
          pl.BlockSpec((1, D), lambda i: (0, 0)),
          pl.BlockSpec((D, D), lambda i: (0, 0)),
          pl.BlockSpec((1, D), lambda i: (0, 0)),
          pl.BlockSpec((NC, _RB, 1), lambda i: (0, i, 0)),
          pl.BlockSpec(memory_space=pltpu.SMEM),
      ],
      out_specs=[
          pl.BlockSpec((_RB, D), lambda i: (i, 0)),
          pl.BlockSpec((_RB, D), lambda i: (i, 0)),
          pl.BlockSpec((_RB, 1), lambda i: (i, 0)),
      ],
      out_shape=[
          jax.ShapeDtypeStruct((N_PAD, D), jnp.float32),
          jax.ShapeDtypeStruct((N_PAD, D), jnp.float32),
          jax.ShapeDtypeStruct((N_PAD, 1), jnp.float32),
      ],
  )(x_pad, W1, b1.reshape(1, D), W2, b2.reshape(1, D), degp3, t0)


def _comb_body(pa_ref, pb_ref, hp_ref, hid_ref, dis_ref, tk_ref,
               hp_out, hid_out):
  s = pa_ref[...] + pb_ref[...] + hp_ref[...]
  hraw = dis_ref[...] * s
  hid_out[...] = hid_ref[...] + tk_ref[0, 0] * hraw
  hp_out[...] = dis_ref[...] * hraw


def _comb_call(pa, pb, hp, hid, dis, tk):
  return pl.pallas_call(
      _comb_body,
      grid=(_GRID,),
      in_specs=[
          pl.BlockSpec((_RB, D), lambda i: (i, 0)),
          pl.BlockSpec((_RB, D), lambda i: (i, 0)),
          pl.BlockSpec((_RB, D), lambda i: (i, 0)),
          pl.BlockSpec((_RB, D), lambda i: (i, 0)),
          pl.BlockSpec((_RB, 1), lambda i: (i, 0)),
          pl.BlockSpec(memory_space=pltpu.SMEM),
      ],
      out_specs=[
          pl.BlockSpec((_RB, D), lambda i: (i, 0)),
          pl.BlockSpec((_RB, D), lambda i: (i, 0)),
      ],
      out_shape=[
          jax.ShapeDtypeStruct((N_PAD, D), jnp.float32),
          jax.ShapeDtypeStruct((N_PAD, D), jnp.float32),
      ],
  )(pa, pb, hp, hid, dis, tk)


def kernel(x, edge_index, edge_weight, W1, b1, W2, b2, temp):
  N, d = x.shape
  E = edge_weight.shape[0]
  K = temp.shape[0] - 1

  src = edge_index[0].astype(jnp.int32)
  dst = edge_index[1].astype(jnp.int32)
  ew = edge_weight.astype(jnp.float32)

  e_per_w = -(-E // (NW * EDGE_B)) * EDGE_B
  e_pad = e_per_w * NW - E
  if e_pad:
    src = jnp.concatenate([src, jnp.zeros((e_pad,), jnp.int32)])
    dst = jnp.concatenate([dst, jnp.zeros((e_pad,), jnp.int32)])
    ew = jnp.concatenate([ew, jnp.zeros((e_pad,), jnp.float32)])

  x_pad = jnp.zeros((N_PAD, D), jnp.float32).at[:N].set(x)

  degp = _make_deg(e_per_w)(dst, ew)
  hp, hid, dis = _mlp_call(x_pad, W1, b1, W2, b2,
                           degp[:, :, :1], temp[0].reshape(1, 1))

  prop = _make_prop(e_per_w)
  for k in range(K):
    parts = prop(hp, src, dst, ew)
    hp, hid = _comb_call(parts[0], parts[1], hp, hid, dis,
                         temp[k + 1].reshape(1, 1))
  return hid[:N]


# speedup vs baseline: 11.6482x; 2.1983x over previous
import functools
import jax
import jax.numpy as jnp
from jax import lax
from jax.experimental import pallas as pl
from jax.experimental.pallas import tpu as pltpu
from jax.experimental.pallas import tpu_sc as plsc

NC = 2
NS = 16
NW = NC * NS
EDGE_B = 128
N_PAD = 10240
D = 128
DH = D // NC
ROWS_PER_TILE = N_PAD // NS

_mesh = functools.partial(
    plsc.VectorSubcoreMesh, core_axis_name="c", subcore_axis_name="s",
    num_cores=NC, num_subcores=NS)


def _wid():
  return lax.axis_index("s") * NC + lax.axis_index("c")


def _make_deg(e_per_w):
  batches = e_per_w // EDGE_B

  DW = 16

  @functools.partial(
      pl.kernel,
      out_type=jax.ShapeDtypeStruct((NC, N_PAD, DW), jnp.float32),
      mesh=_mesh(),
      scratch_types=[
          pltpu.VMEM((EDGE_B,), jnp.int32),
          pltpu.VMEM((EDGE_B,), jnp.float32),
          pltpu.VMEM((EDGE_B, DW), jnp.float32),
          pltpu.VMEM_SHARED((N_PAD, DW), jnp.float32),
      ],
  )
  def k(dst_hbm, ew_hbm, out_hbm, idx_v, ew_v, ew_col, deg_sh):
    cid = lax.axis_index("c")
    sid = lax.axis_index("s")
    wid = _wid()
    row0 = sid * ROWS_PER_TILE

    def zb(i, _):
      ew_col[i, pl.ds(0, 16)] = jnp.zeros((16,), jnp.float32)
      return 0
    lax.fori_loop(0, EDGE_B, zb, 0)
    for t in range(ROWS_PER_TILE // EDGE_B):
      pltpu.sync_copy(ew_col, deg_sh.at[pl.ds(row0 + t * EDGE_B, EDGE_B)])
    plsc.subcore_barrier()

    def ebody(b, _):
      ebase = wid * e_per_w + b * EDGE_B
      pltpu.sync_copy(dst_hbm.at[pl.ds(ebase, EDGE_B)], idx_v)
      pltpu.sync_copy(ew_hbm.at[pl.ds(ebase, EDGE_B)], ew_v)

      def sbody(g, _):
        w16 = ew_v[pl.ds(g * 16, 16)]
        base = g * 16
        for l in range(16):
          ew_col[base + l, pl.ds(0, 16)] = jnp.full((16,), w16[l],
                                                    jnp.float32)
        return 0
      lax.fori_loop(0, EDGE_B // 16, sbody, 0)

      pltpu.sync_copy(ew_col, deg_sh.at[idx_v], add=True)
      return 0
    lax.fori_loop(0, batches, ebody, 0)

    plsc.subcore_barrier()
    pltpu.sync_copy(deg_sh.at[pl.ds(row0, ROWS_PER_TILE)],
                    out_hbm.at[cid, pl.ds(row0, ROWS_PER_TILE)])

  return k


def _make_prop(e_per_w):
  batches = e_per_w // EDGE_B
  CB = 16
  nchunks = batches // CB

  @functools.partial(
      pl.kernel,
      out_type=jax.ShapeDtypeStruct((NC, N_PAD, D), jnp.float32),
      mesh=_mesh(),
      scratch_types=[
          pltpu.VMEM((CB * EDGE_B,), jnp.int32),
          pltpu.VMEM((CB * EDGE_B,), jnp.int32),
          pltpu.VMEM((CB * EDGE_B,), jnp.float32),
          pltpu.VMEM((EDGE_B, D), jnp.float32),
          pltpu.VMEM((EDGE_B, D), jnp.float32),
          pltpu.VMEM((EDGE_B,), jnp.int32),
          pltpu.VMEM((EDGE_B,), jnp.int32),
          pltpu.VMEM((EDGE_B,), jnp.int32),
          pltpu.VMEM_SHARED((N_PAD, D), jnp.float32),
          pltpu.SemaphoreType.DMA,
          pltpu.SemaphoreType.DMA,
      ],
  )
  def k(hp_hbm, src_hbm, dst_hbm, ew_hbm, out_hbm,
        src2, dst2, ew2, rows_a, rows_b, sidx_a, sidx_b, didx,
        acc_sh, gsem_a, gsem_b):
    cid = lax.axis_index("c")
    sid = lax.axis_index("s")
    wid = _wid()
    row0 = sid * ROWS_PER_TILE

    def zb(i, _):
      for j in range(D // 16):
        rows_a[i, pl.ds(j * 16, 16)] = jnp.zeros((16,), jnp.float32)
      return 0
    lax.fori_loop(0, EDGE_B, zb, 0)
    for t in range(ROWS_PER_TILE // EDGE_B):
      pltpu.sync_copy(rows_a, acc_sh.at[pl.ds(row0 + t * EDGE_B, EDGE_B)])
    plsc.subcore_barrier()

    def gather(b, buf, sidx, sem):
      for g in range(EDGE_B // 16):
        sidx[pl.ds(g * 16, 16)] = src2[pl.ds(b * EDGE_B + g * 16, 16)]
      pltpu.async_copy(hp_hbm.at[sidx], buf, sem)

    def gwait(buf, sidx, sem):
      pltpu.make_async_copy(hp_hbm.at[sidx], buf, sem).wait()

    def scale(b, buf):
      def sbody(g, _):
        w16 = ew2[pl.ds(b * EDGE_B + g * 16, 16)]
        base = g * 16
        for l in range(16):
          w = w16[l]
          for j in range(D // 16):
            buf[base + l, pl.ds(j * 16, 16)] = (
                buf[base + l, pl.ds(j * 16, 16)] * w)
        return 0
      lax.fori_loop(0, EDGE_B // 16, sbody, 0)

    def scat(b, buf):
      for g in range(EDGE_B // 16):
        didx[pl.ds(g * 16, 16)] = dst2[pl.ds(b * EDGE_B + g * 16, 16)]
      pltpu.sync_copy(buf, acc_sh.at[didx], add=True)

    def chunk(c, _):
      ebase = pl.multiple_of((wid * batches + c * CB) * EDGE_B, 8)
      pltpu.sync_copy(src_hbm.at[pl.ds(ebase, CB * EDGE_B)], src2)
      pltpu.sync_copy(dst_hbm.at[pl.ds(ebase, CB * EDGE_B)], dst2)
      pltpu.sync_copy(ew_hbm.at[pl.ds(ebase, CB * EDGE_B)], ew2)

      gather(0, rows_a, sidx_a, gsem_a)

      def pair(p, _):
        b0 = 2 * p
        gwait(rows_a, sidx_a, gsem_a)
        gather(b0 + 1, rows_b, sidx_b, gsem_b)
        scale(b0, rows_a)
        scat(b0, rows_a)
        gwait(rows_b, sidx_b, gsem_b)

        @pl.when(p < CB // 2 - 1)
        def _():
          gather(b0 + 2, rows_a, sidx_a, gsem_a)
        scale(b0 + 1, rows_b)
        scat(b0 + 1, rows_b)
        return 0
      lax.fori_loop(0, CB // 2, pair, 0)
      return 0
    lax.fori_loop(0, nchunks, chunk, 0)

    plsc.subcore_barrier()
    pltpu.sync_copy(acc_sh.at[pl.ds(row0, ROWS_PER_TILE)],
                    out_hbm.at[cid, pl.ds(row0, ROWS_PER_TILE)])

  return k


_RB = 1024
_GRID_N = N_PAD // _RB


def _mlp_body(x_ref, w1_ref, b1_ref, w2_ref, b2_ref, degp_ref, t0_ref,
              hp_ref, hid_ref, dis_ref):
  xb = x_ref[...]
  h = jnp.maximum(jnp.dot(xb, w1_ref[...],
                          preferred_element_type=jnp.float32) + b1_ref[...],
                  0.0)
  h = jnp.dot(h, w2_ref[...], preferred_element_type=jnp.float32) + b2_ref[...]
  deg = jnp.sum(degp_ref[...], axis=0) + 1.0
  dis = lax.rsqrt(deg)
  hid_ref[...] = t0_ref[0, 0] * h
  hp_ref[...] = dis * h
  dis_ref[...] = dis


def _mlp_call(x_pad, W1, b1, W2, b2, degp3, t0):
  return pl.pallas_call(
      _mlp_body,
      grid=(_GRID_N,),
      in_specs=[
          pl.BlockSpec((_RB, D), lambda i: (i, 0)),
          pl.BlockSpec((D, D), lambda i: (0, 0)),
          pl.BlockSpec((1, D), lambda i: (0, 0)),
          pl.BlockSpec((D, D), lambda i: (0, 0)),
          pl.BlockSpec((1, D), lambda i: (0, 0)),
          pl.BlockSpec((NC, _RB, 1), lambda i: (0, i, 0)),
          pl.BlockSpec(memory_space=pltpu.SMEM),
      ],
      out_specs=[
          pl.BlockSpec((_RB, D), lambda i: (i, 0)),
          pl.BlockSpec((_RB, D), lambda i: (i, 0)),
          pl.BlockSpec((_RB, 1), lambda i: (i, 0)),
      ],
      out_shape=[
          jax.ShapeDtypeStruct((N_PAD, D), jnp.float32),
          jax.ShapeDtypeStruct((N_PAD, D), jnp.float32),
          jax.ShapeDtypeStruct((N_PAD, 1), jnp.float32),
      ],
  )(x_pad, W1, b1.reshape(1, D), W2, b2.reshape(1, D), degp3, t0)


def _comb_body(pa_ref, pb_ref, hp_ref, hid_ref, dis_ref, tk_ref,
               hp_out, hid_out):
  s = pa_ref[...] + pb_ref[...] + hp_ref[...]
  hraw = dis_ref[...] * s
  hid_out[...] = hid_ref[...] + tk_ref[0, 0] * hraw
  hp_out[...] = dis_ref[...] * hraw


def _comb_call(pa, pb, hp, hid, dis, tk):
  return pl.pallas_call(
      _comb_body,
      grid=(_GRID_N,),
      in_specs=[
          pl.BlockSpec((_RB, D), lambda i: (i, 0)),
          pl.BlockSpec((_RB, D), lambda i: (i, 0)),
          pl.BlockSpec((_RB, D), lambda i: (i, 0)),
          pl.BlockSpec((_RB, D), lambda i: (i, 0)),
          pl.BlockSpec((_RB, 1), lambda i: (i, 0)),
          pl.BlockSpec(memory_space=pltpu.SMEM),
      ],
      out_specs=[
          pl.BlockSpec((_RB, D), lambda i: (i, 0)),
          pl.BlockSpec((_RB, D), lambda i: (i, 0)),
      ],
      out_shape=[
          jax.ShapeDtypeStruct((N_PAD, D), jnp.float32),
          jax.ShapeDtypeStruct((N_PAD, D), jnp.float32),
      ],
  )(pa, pb, hp, hid, dis, tk)


def kernel(x, edge_index, edge_weight, W1, b1, W2, b2, temp):
  N, _ = x.shape
  E = edge_weight.shape[0]
  K = temp.shape[0] - 1

  src = edge_index[0].astype(jnp.int32)
  dst = edge_index[1].astype(jnp.int32)
  ew = edge_weight.astype(jnp.float32)

  e_per_w = -(-E // (NW * EDGE_B)) * EDGE_B
  e_pad = e_per_w * NW - E
  if e_pad:
    src = jnp.concatenate([src, jnp.zeros((e_pad,), jnp.int32)])
    dst = jnp.concatenate([dst, jnp.zeros((e_pad,), jnp.int32)])
    ew = jnp.concatenate([ew, jnp.zeros((e_pad,), jnp.float32)])
  x_pad = jnp.zeros((N_PAD, D), jnp.float32).at[:N].set(x)
  degp = _make_deg(e_per_w)(dst, ew)[:, :, :1]
  hp, hid, dis = _mlp_call(x_pad, W1, b1, W2, b2,
                           degp, temp[0].reshape(1, 1))

  prop = _make_prop(e_per_w)
  for k in range(K):
    parts = prop(hp, src, dst, ew)
    hp, hid = _comb_call(parts[0], parts[1], hp, hid, dis,
                         temp[k + 1].reshape(1, 1))

  return hid[:N]


# speedup vs baseline: 11.6556x; 1.0006x over previous
import functools
import jax
import jax.numpy as jnp
from jax import lax
from jax.experimental import pallas as pl
from jax.experimental.pallas import tpu as pltpu
from jax.experimental.pallas import tpu_sc as plsc

NC = 2
NS = 16
NW = NC * NS
EDGE_B = 128
N_PAD = 10240
D = 128
DH = D // NC
ROWS_PER_TILE = N_PAD // NS

_mesh = functools.partial(
    plsc.VectorSubcoreMesh, core_axis_name="c", subcore_axis_name="s",
    num_cores=NC, num_subcores=NS)


def _wid():
  return lax.axis_index("s") * NC + lax.axis_index("c")


def _make_deg(e_per_w):
  batches = e_per_w // EDGE_B

  DW = 16

  @functools.partial(
      pl.kernel,
      out_type=jax.ShapeDtypeStruct((NC, N_PAD, DW), jnp.float32),
      mesh=_mesh(),
      scratch_types=[
          pltpu.VMEM((EDGE_B,), jnp.int32),
          pltpu.VMEM((EDGE_B,), jnp.float32),
          pltpu.VMEM((EDGE_B, DW), jnp.float32),
          pltpu.VMEM_SHARED((N_PAD, DW), jnp.float32),
      ],
  )
  def k(dst_hbm, ew_hbm, out_hbm, idx_v, ew_v, ew_col, deg_sh):
    cid = lax.axis_index("c")
    sid = lax.axis_index("s")
    wid = _wid()
    row0 = sid * ROWS_PER_TILE

    def zb(i, _):
      ew_col[i, pl.ds(0, 16)] = jnp.zeros((16,), jnp.float32)
      return 0
    lax.fori_loop(0, EDGE_B, zb, 0)
    for t in range(ROWS_PER_TILE // EDGE_B):
      pltpu.sync_copy(ew_col, deg_sh.at[pl.ds(row0 + t * EDGE_B, EDGE_B)])
    plsc.subcore_barrier()

    def ebody(b, _):
      ebase = wid * e_per_w + b * EDGE_B
      pltpu.sync_copy(dst_hbm.at[pl.ds(ebase, EDGE_B)], idx_v)
      pltpu.sync_copy(ew_hbm.at[pl.ds(ebase, EDGE_B)], ew_v)

      def sbody(g, _):
        w16 = ew_v[pl.ds(g * 16, 16)]
        base = g * 16
        for l in range(16):
          ew_col[base + l, pl.ds(0, 16)] = jnp.full((16,), w16[l],
                                                    jnp.float32)
        return 0
      lax.fori_loop(0, EDGE_B // 16, sbody, 0)

      pltpu.sync_copy(ew_col, deg_sh.at[idx_v], add=True)
      return 0
    lax.fori_loop(0, batches, ebody, 0)

    plsc.subcore_barrier()
    pltpu.sync_copy(deg_sh.at[pl.ds(row0, ROWS_PER_TILE)],
                    out_hbm.at[cid, pl.ds(row0, ROWS_PER_TILE)])

  return k


def _make_prop(e_per_w):
  batches = e_per_w // EDGE_B
  CB = 16
  nchunks = batches // CB

  @functools.partial(
      pl.kernel,
      out_type=jax.ShapeDtypeStruct((NC, N_PAD, D), jnp.float32),
      mesh=_mesh(),
      scratch_types=[
          pltpu.VMEM((CB * EDGE_B,), jnp.int32),
          pltpu.VMEM((CB * EDGE_B,), jnp.int32),
          pltpu.VMEM((CB * EDGE_B,), jnp.float32),
          pltpu.VMEM((EDGE_B, D), jnp.float32),
          pltpu.VMEM((EDGE_B, D), jnp.float32),
          pltpu.VMEM((EDGE_B,), jnp.int32),
          pltpu.VMEM((EDGE_B,), jnp.int32),
          pltpu.VMEM((EDGE_B,), jnp.int32),
          pltpu.VMEM((EDGE_B,), jnp.int32),
          pltpu.VMEM_SHARED((N_PAD, D), jnp.float32),
          pltpu.SemaphoreType.DMA,
          pltpu.SemaphoreType.DMA,
          pltpu.SemaphoreType.DMA,
          pltpu.SemaphoreType.DMA,
      ],
  )
  def k(hp_hbm, src_hbm, dst_hbm, ew_hbm, out_hbm,
        src2, dst2, ew2, rows_a, rows_b, sidx_a, sidx_b, didx_a, didx_b,
        acc_sh, gsem_a, gsem_b, ssem_a, ssem_b):
    cid = lax.axis_index("c")
    sid = lax.axis_index("s")
    wid = _wid()
    row0 = sid * ROWS_PER_TILE

    def zb(i, _):
      for j in range(D // 16):
        rows_a[i, pl.ds(j * 16, 16)] = jnp.zeros((16,), jnp.float32)
      return 0
    lax.fori_loop(0, EDGE_B, zb, 0)
    for t in range(ROWS_PER_TILE // EDGE_B):
      pltpu.sync_copy(rows_a, acc_sh.at[pl.ds(row0 + t * EDGE_B, EDGE_B)])
    plsc.subcore_barrier()

    def gather(b, buf, sidx, sem):
      for g in range(EDGE_B // 16):
        sidx[pl.ds(g * 16, 16)] = src2[pl.ds(b * EDGE_B + g * 16, 16)]
      pltpu.async_copy(hp_hbm.at[sidx], buf, sem)

    def gwait(buf, sidx, sem):
      pltpu.make_async_copy(hp_hbm.at[sidx], buf, sem).wait()

    def scale(b, buf):
      def sbody(g, _):
        w16 = ew2[pl.ds(b * EDGE_B + g * 16, 16)]
        base = g * 16
        for l in range(16):
          w = w16[l]
          for j in range(D // 16):
            buf[base + l, pl.ds(j * 16, 16)] = (
                buf[base + l, pl.ds(j * 16, 16)] * w)
        return 0
      lax.fori_loop(0, EDGE_B // 16, sbody, 0)

    def scat(b, buf, didx, sem):
      for g in range(EDGE_B // 16):
        didx[pl.ds(g * 16, 16)] = dst2[pl.ds(b * EDGE_B + g * 16, 16)]
      pltpu.async_copy(buf, acc_sh.at[didx], sem, add=True)

    def swait(buf, didx, sem):
      pltpu.make_async_copy(buf, acc_sh.at[didx], sem).wait()

    def chunk(c, _):
      ebase = pl.multiple_of((wid * batches + c * CB) * EDGE_B, 8)
      pltpu.sync_copy(src_hbm.at[pl.ds(ebase, CB * EDGE_B)], src2)
      pltpu.sync_copy(dst_hbm.at[pl.ds(ebase, CB * EDGE_B)], dst2)
      pltpu.sync_copy(ew_hbm.at[pl.ds(ebase, CB * EDGE_B)], ew2)

      gather(0, rows_a, sidx_a, gsem_a)

      def pair(p, _):
        b0 = 2 * p
        gwait(rows_a, sidx_a, gsem_a)

        @pl.when(p > 0)
        def _():
          swait(rows_b, didx_b, ssem_b)
        gather(b0 + 1, rows_b, sidx_b, gsem_b)
        scale(b0, rows_a)
        scat(b0, rows_a, didx_a, ssem_a)
        gwait(rows_b, sidx_b, gsem_b)

        @pl.when(p < CB // 2 - 1)
        def _():
          swait(rows_a, didx_a, ssem_a)
          gather(b0 + 2, rows_a, sidx_a, gsem_a)
        scale(b0 + 1, rows_b)
        scat(b0 + 1, rows_b, didx_b, ssem_b)
        return 0
      lax.fori_loop(0, CB // 2, pair, 0)
      swait(rows_a, didx_a, ssem_a)
      swait(rows_b, didx_b, ssem_b)
      return 0
    lax.fori_loop(0, nchunks, chunk, 0)

    plsc.subcore_barrier()
    pltpu.sync_copy(acc_sh.at[pl.ds(row0, ROWS_PER_TILE)],
                    out_hbm.at[cid, pl.ds(row0, ROWS_PER_TILE)])

  return k


_RB = 1024
_GRID_N = N_PAD // _RB


def _mlp_body(x_ref, w1_ref, b1_ref, w2_ref, b2_ref, degp_ref, t0_ref,
              hp_ref, hid_ref, dis_ref):
  xb = x_ref[...]
  h = jnp.maximum(jnp.dot(xb, w1_ref[...],
                          preferred_element_type=jnp.float32) + b1_ref[...],
                  0.0)
  h = jnp.dot(h, w2_ref[...], preferred_element_type=jnp.float32) + b2_ref[...]
  deg = jnp.sum(degp_ref[...], axis=0) + 1.0
  dis = lax.rsqrt(deg)
  hid_ref[...] = t0_ref[0, 0] * h
  hp_ref[...] = dis * h
  dis_ref[...] = dis


def _mlp_call(x_pad, W1, b1, W2, b2, degp3, t0):
  return pl.pallas_call(
      _mlp_body,
      grid=(_GRID_N,),
      in_specs=[
          pl.BlockSpec((_RB, D), lambda i: (i, 0)),
          pl.BlockSpec((D, D), lambda i: (0, 0)),
          pl.BlockSpec((1, D), lambda i: (0, 0)),
          pl.BlockSpec((D, D), lambda i: (0, 0)),
          pl.BlockSpec((1, D), lambda i: (0, 0)),
          pl.BlockSpec((NC, _RB, 1), lambda i: (0, i, 0)),
          pl.BlockSpec(memory_space=pltpu.SMEM),
      ],
      out_specs=[
          pl.BlockSpec((_RB, D), lambda i: (i, 0)),
          pl.BlockSpec((_RB, D), lambda i: (i, 0)),
          pl.BlockSpec((_RB, 1), lambda i: (i, 0)),
      ],
      out_shape=[
          jax.ShapeDtypeStruct((N_PAD, D), jnp.float32),
          jax.ShapeDtypeStruct((N_PAD, D), jnp.float32),
          jax.ShapeDtypeStruct((N_PAD, 1), jnp.float32),
      ],
  )(x_pad, W1, b1.reshape(1, D), W2, b2.reshape(1, D), degp3, t0)


def _comb_body(pa_ref, pb_ref, hp_ref, hid_ref, dis_ref, tk_ref,
               hp_out, hid_out):
  s = pa_ref[...] + pb_ref[...] + hp_ref[...]
  hraw = dis_ref[...] * s
  hid_out[...] = hid_ref[...] + tk_ref[0, 0] * hraw
  hp_out[...] = dis_ref[...] * hraw


def _comb_call(pa, pb, hp, hid, dis, tk):
  return pl.pallas_call(
      _comb_body,
      grid=(_GRID_N,),
      in_specs=[
          pl.BlockSpec((_RB, D), lambda i: (i, 0)),
          pl.BlockSpec((_RB, D), lambda i: (i, 0)),
          pl.BlockSpec((_RB, D), lambda i: (i, 0)),
          pl.BlockSpec((_RB, D), lambda i: (i, 0)),
          pl.BlockSpec((_RB, 1), lambda i: (i, 0)),
          pl.BlockSpec(memory_space=pltpu.SMEM),
      ],
      out_specs=[
          pl.BlockSpec((_RB, D), lambda i: (i, 0)),
          pl.BlockSpec((_RB, D), lambda i: (i, 0)),
      ],
      out_shape=[
          jax.ShapeDtypeStruct((N_PAD, D), jnp.float32),
          jax.ShapeDtypeStruct((N_PAD, D), jnp.float32),
      ],
  )(pa, pb, hp, hid, dis, tk)


def kernel(x, edge_index, edge_weight, W1, b1, W2, b2, temp):
  N, _ = x.shape
  E = edge_weight.shape[0]
  K = temp.shape[0] - 1

  src = edge_index[0].astype(jnp.int32)
  dst = edge_index[1].astype(jnp.int32)
  ew = edge_weight.astype(jnp.float32)

  e_per_w = -(-E // (NW * EDGE_B)) * EDGE_B
  e_pad = e_per_w * NW - E
  if e_pad:
    src = jnp.concatenate([src, jnp.zeros((e_pad,), jnp.int32)])
    dst = jnp.concatenate([dst, jnp.zeros((e_pad,), jnp.int32)])
    ew = jnp.concatenate([ew, jnp.zeros((e_pad,), jnp.float32)])
  x_pad = jnp.zeros((N_PAD, D), jnp.float32).at[:N].set(x)
  degp = _make_deg(e_per_w)(dst, ew)[:, :, :1]
  hp, hid, dis = _mlp_call(x_pad, W1, b1, W2, b2,
                           degp, temp[0].reshape(1, 1))

  prop = _make_prop(e_per_w)
  for k in range(K):
    parts = prop(hp, src, dst, ew)
    hp, hid = _comb_call(parts[0], parts[1], hp, hid, dis,
                         temp[k + 1].reshape(1, 1))

  return hid[:N]
